# baseline (speedup 1.0000x reference)
import functools

import jax
import jax.numpy as jnp
from jax import lax
from jax.experimental import pallas as pl
from jax.experimental.pallas import tpu as pltpu
from jax.experimental.pallas import tpu_sc as plsc

_B, _N, _C, _H, _W = 4, 4096, 384, 224, 224
_HW = _H * _W
_L = 16
_NWORKERS = 32
_WPB = _NWORKERS // _B
_NCH = _C // _WPB
_NCHUNK = _N // _L

_MIN_XY = -51.2
_SCALE = (_W - 1) / 102.4


def _prep(v):
    vc = jnp.clip(v, -8.0, 232.0)
    t = vc.astype(jnp.int32).astype(jnp.float32)
    f0 = jnp.where(t > vc, t - 1.0, t)
    w1 = vc - f0
    a0 = jnp.where((f0 >= 0.0) & (f0 <= _W - 1.0), 1.0 - w1, 0.0)
    a1 = jnp.where((f0 >= -1.0) & (f0 <= _W - 2.0), w1, 0.0)
    i0 = jnp.clip(f0, 0.0, _W - 1.0).astype(jnp.int32)
    i1 = jnp.clip(f0 + 1.0, 0.0, _W - 1.0).astype(jnp.int32)
    return a0, a1, i0, i1


@functools.partial(
    pl.kernel,
    mesh=plsc.VectorSubcoreMesh(core_axis_name="c", subcore_axis_name="s"),
    compiler_params=pltpu.CompilerParams(needs_layout_passes=False),
    out_type=jax.ShapeDtypeStruct((_B, _C, _N), jnp.float32),
    scratch_types=[
        pltpu.VMEM((_N,), jnp.float32),
        pltpu.VMEM((_N,), jnp.float32),
        pltpu.VMEM((_N,), jnp.int32),
        pltpu.VMEM((_N,), jnp.int32),
        pltpu.VMEM((_N,), jnp.int32),
        pltpu.VMEM((_N,), jnp.int32),
        pltpu.VMEM((_N,), jnp.float32),
        pltpu.VMEM((_N,), jnp.float32),
        pltpu.VMEM((_N,), jnp.float32),
        pltpu.VMEM((_N,), jnp.float32),
        pltpu.VMEM((_HW,), jnp.float32),
        pltpu.VMEM((_N,), jnp.float32),
    ],
)
def _sc_sample(xs_hbm, ys_hbm, feat_hbm, out_hbm,
               xs_v, ys_v, i00_v, i01_v, i10_v, i11_v,
               ax0_v, ax1_v, ay0_v, ay1_v, fmap_v, row_v):
    cid = lax.axis_index("c")
    sid = lax.axis_index("s")
    wid = sid * 2 + cid
    b = wid // _WPB
    c0 = (wid % _WPB) * _NCH

    pltpu.sync_copy(xs_hbm.at[b], xs_v)
    pltpu.sync_copy(ys_hbm.at[b], ys_v)

    def precompute(i, carry):
        sl = pl.ds(i * _L, _L)
        ix = (xs_v[sl] - _MIN_XY) * _SCALE
        iy = (ys_v[sl] - _MIN_XY) * _SCALE
        ax0, ax1, xi0, xi1 = _prep(ix)
        ay0, ay1, yi0, yi1 = _prep(iy)
        r0 = yi0 * _W
        r1 = yi1 * _W
        i00_v[sl] = r0 + xi0
        i01_v[sl] = r0 + xi1
        i10_v[sl] = r1 + xi0
        i11_v[sl] = r1 + xi1
        ax0_v[sl] = ax0
        ax1_v[sl] = ax1
        ay0_v[sl] = ay0
        ay1_v[sl] = ay1
        return carry

    lax.fori_loop(0, _NCHUNK, precompute, 0)

    def per_channel(j, carry):
        c = c0 + j
        pltpu.sync_copy(feat_hbm.at[b, c], fmap_v)

        def chunk(i, inner):
            sl = pl.ds(i * _L, _L)
            g00 = plsc.load_gather(fmap_v, [i00_v[sl]])
            g01 = plsc.load_gather(fmap_v, [i01_v[sl]])
            g10 = plsc.load_gather(fmap_v, [i10_v[sl]])
            g11 = plsc.load_gather(fmap_v, [i11_v[sl]])
            ax0 = ax0_v[sl]
            ax1 = ax1_v[sl]
            row_v[sl] = (ay0_v[sl] * (ax0 * g00 + ax1 * g01)
                         + ay1_v[sl] * (ax0 * g10 + ax1 * g11))
            return inner

        lax.fori_loop(0, _NCHUNK, chunk, 0)
        pltpu.sync_copy(row_v, out_hbm.at[b, c])
        return carry

    lax.fori_loop(0, _NCH, per_channel, 0)


def kernel(boxes_3d, bev_features):
    assert boxes_3d.shape == (_B, _N, 7)
    assert bev_features.shape == (_B, _C, _H, _W)
    xs = boxes_3d[:, :, 0]
    ys = boxes_3d[:, :, 1]
    feat = bev_features.reshape(_B, _C, _HW)
    out_cn = _sc_sample(xs, ys, feat)
    roi_features = jnp.transpose(out_cn, (0, 2, 1))
    rois = jnp.concatenate(
        [boxes_3d[:, :, 0:2], boxes_3d[:, :, 3:5], boxes_3d[:, :, 6:7]],
        axis=-1)
    return (rois, roi_features)
